# baseline (speedup 1.0000x reference)
import functools

import jax
import jax.numpy as jnp
from jax import lax
from jax.experimental import pallas as pl
from jax.experimental.pallas import tpu as pltpu
from jax.experimental.pallas import tpu_sc as plsc

NUM_ROWS = 1000000
BATCH = 16384
EMBED = 64
HIDDEN = 128

_NC, _NS = 2, 16
_NW = _NC * _NS
_B_PER_W = BATCH // _NW
_E_PER_W = _B_PER_W * EMBED
_CH = 128
_NCHUNK = _E_PER_W // _CH


def _gather_body(fidx_u_hbm, fidx_b_hbm, uflat_hbm, bflat_hbm,
                 ue_out, be_out, idx_v, vals_v, sem):
    wid = lax.axis_index("s") * _NC + lax.axis_index("c")
    base = pl.multiple_of(wid * _E_PER_W, _E_PER_W)

    def one_table(fidx_hbm, flat_hbm, out_hbm):
        pltpu.sync_copy(fidx_hbm.at[pl.ds(base, _E_PER_W)], idx_v)

        def chunk(j, c):
            off = pl.multiple_of(j * _CH, _CH)
            pltpu.make_async_copy(
                flat_hbm.at[idx_v.at[pl.ds(off, _CH)]],
                vals_v.at[pl.ds(off, _CH)], sem).start()
            return c

        lax.fori_loop(0, _NCHUNK, chunk, 0)
        pltpu.make_async_copy(flat_hbm.at[pl.ds(0, _E_PER_W)], vals_v,
                              sem).wait()
        pltpu.sync_copy(vals_v, out_hbm.at[pl.ds(base, _E_PER_W)])

    one_table(fidx_u_hbm, uflat_hbm, ue_out)
    one_table(fidx_b_hbm, bflat_hbm, be_out)


@functools.lru_cache(maxsize=1)
def _make_gather():
    return pl.kernel(
        _gather_body,
        mesh=plsc.VectorSubcoreMesh(core_axis_name="c", subcore_axis_name="s"),
        out_type=[
            jax.ShapeDtypeStruct((BATCH * EMBED,), jnp.float32),
            jax.ShapeDtypeStruct((BATCH * EMBED,), jnp.float32),
        ],
        scratch_types=[
            pltpu.VMEM((_E_PER_W,), jnp.int32),
            pltpu.VMEM((_E_PER_W,), jnp.float32),
            pltpu.SemaphoreType.DMA,
        ],
    )


_BS = 2048


def _mlp_body(ue_ref, be_ref, w1t_ref, b1_ref, w2t_ref, b2_ref, out_ref):
    w1t = w1t_ref[:]
    h = (jnp.dot(ue_ref[:], w1t[:EMBED], preferred_element_type=jnp.float32)
         + jnp.dot(be_ref[:], w1t[EMBED:], preferred_element_type=jnp.float32)
         + b1_ref[:])
    h = jnp.where(h >= 0, h, 0.01 * h)
    raw = jnp.dot(h, w2t_ref[:], preferred_element_type=jnp.float32) + b2_ref[0, 0]
    out_ref[:] = 1.0 + 4.0 * jax.nn.sigmoid(raw)


_mlp = pl.pallas_call(
    _mlp_body,
    grid=(BATCH // _BS,),
    in_specs=[
        pl.BlockSpec((_BS, EMBED), lambda i: (i, 0)),
        pl.BlockSpec((_BS, EMBED), lambda i: (i, 0)),
        pl.BlockSpec((HIDDEN, HIDDEN), lambda i: (0, 0)),
        pl.BlockSpec((1, HIDDEN), lambda i: (0, 0)),
        pl.BlockSpec((HIDDEN, 1), lambda i: (0, 0)),
        pl.BlockSpec(memory_space=pltpu.SMEM),
    ],
    out_specs=pl.BlockSpec((_BS, 1), lambda i: (i, 0)),
    out_shape=jax.ShapeDtypeStruct((BATCH, 1), jnp.float32),
)


def kernel(user_id, book_id, user_emb, book_emb, W1, b1, W2, b2):
    uid = user_id.astype(jnp.int32)
    bid = book_id.astype(jnp.int32)
    d_off = (jnp.arange(EMBED, dtype=jnp.int32) * NUM_ROWS)[None, :]
    fidx_u = (uid[:, None] + d_off).reshape(-1)
    fidx_b = (bid[:, None] + d_off).reshape(-1)
    uflat = user_emb.swapaxes(0, 1).reshape(-1)
    bflat = book_emb.swapaxes(0, 1).reshape(-1)
    ue_flat, be_flat = _make_gather()(fidx_u, fidx_b, uflat, bflat)
    ue = ue_flat.reshape(BATCH, EMBED)
    be = be_flat.reshape(BATCH, EMBED)
    return _mlp(ue, be, W1.swapaxes(0, 1), b1.reshape(1, HIDDEN),
                W2.swapaxes(0, 1), b2.reshape(1, 1))


# speedup vs baseline: 9.7886x; 9.7886x over previous
import jax
import jax.numpy as jnp
from jax import lax
from jax.experimental import pallas as pl
from jax.experimental.pallas import tpu as pltpu

NUM_ROWS = 1000000
BATCH = 16384
EMBED = 64
HIDDEN = 128

_GBS = 1024
_WIN = 32


def _gather_body(uid_ref, bid_ref, utbl, btbl, out_u, out_b, usem, bsem):
    i = pl.program_id(0)
    base = i * _GBS

    def start_one(k):
        u = uid_ref[base + k]
        b = bid_ref[base + k]
        pltpu.make_async_copy(
            utbl.at[pl.ds(u, 1)], out_u.at[pl.ds(k, 1)], usem).start()
        pltpu.make_async_copy(
            btbl.at[pl.ds(b, 1)], out_b.at[pl.ds(k, 1)], bsem).start()

    def wait_one():
        pltpu.make_async_copy(
            utbl.at[pl.ds(0, 1)], out_u.at[pl.ds(0, 1)], usem).wait()
        pltpu.make_async_copy(
            btbl.at[pl.ds(0, 1)], out_b.at[pl.ds(0, 1)], bsem).wait()

    lax.fori_loop(0, _WIN, lambda k, c: (start_one(k), c)[1], 0)

    def steady(k, c):
        wait_one()
        start_one(k + _WIN)
        return c

    lax.fori_loop(0, _GBS - _WIN, steady, 0)
    lax.fori_loop(0, _WIN, lambda k, c: (wait_one(), c)[1], 0)


_gather = pl.pallas_call(
    _gather_body,
    grid_spec=pltpu.PrefetchScalarGridSpec(
        num_scalar_prefetch=2,
        grid=(BATCH // _GBS,),
        in_specs=[
            pl.BlockSpec(memory_space=pl.ANY),
            pl.BlockSpec(memory_space=pl.ANY),
        ],
        out_specs=[
            pl.BlockSpec((_GBS, EMBED), lambda i, uid, bid: (i, 0)),
            pl.BlockSpec((_GBS, EMBED), lambda i, uid, bid: (i, 0)),
        ],
        scratch_shapes=[pltpu.SemaphoreType.DMA, pltpu.SemaphoreType.DMA],
    ),
    out_shape=[
        jax.ShapeDtypeStruct((BATCH, EMBED), jnp.float32),
        jax.ShapeDtypeStruct((BATCH, EMBED), jnp.float32),
    ],
)

_BS = 2048


def _mlp_body(gu_ref, gb_ref, w1t_ref, b1_ref, w2t_ref, b2_ref, out_ref):
    w1t = w1t_ref[:]
    h = (jnp.dot(gu_ref[:], w1t[:EMBED], preferred_element_type=jnp.float32)
         + jnp.dot(gb_ref[:], w1t[EMBED:], preferred_element_type=jnp.float32)
         + b1_ref[:])
    h = jnp.where(h >= 0, h, 0.01 * h)
    raw = jnp.dot(h, w2t_ref[:], preferred_element_type=jnp.float32) + b2_ref[0, 0]
    out_ref[:] = 1.0 + 4.0 * jax.nn.sigmoid(raw)


_mlp = pl.pallas_call(
    _mlp_body,
    grid=(BATCH // _BS,),
    in_specs=[
        pl.BlockSpec((_BS, EMBED), lambda i: (i, 0)),
        pl.BlockSpec((_BS, EMBED), lambda i: (i, 0)),
        pl.BlockSpec((HIDDEN, HIDDEN), lambda i: (0, 0)),
        pl.BlockSpec((1, HIDDEN), lambda i: (0, 0)),
        pl.BlockSpec((HIDDEN, 1), lambda i: (0, 0)),
        pl.BlockSpec(memory_space=pltpu.SMEM),
    ],
    out_specs=pl.BlockSpec((_BS, 1), lambda i: (i, 0)),
    out_shape=jax.ShapeDtypeStruct((BATCH, 1), jnp.float32),
)


def kernel(user_id, book_id, user_emb, book_emb, W1, b1, W2, b2):
    uid = user_id.astype(jnp.int32)
    bid = book_id.astype(jnp.int32)
    gu, gb = _gather(uid, bid, user_emb, book_emb)
    return _mlp(gu, gb, W1.swapaxes(0, 1), b1.reshape(1, HIDDEN),
                W2.swapaxes(0, 1), b2.reshape(1, 1))


# speedup vs baseline: 11.7757x; 1.2030x over previous
import jax
import jax.numpy as jnp
from jax import lax
from jax.experimental import pallas as pl
from jax.experimental.pallas import tpu as pltpu

NUM_ROWS = 1000000
BATCH = 16384
EMBED = 64
HIDDEN = 128

_GBS = 2048
_WIN = 64


def _body(uid_ref, bid_ref, utbl, btbl, w1t_ref, b1_ref, w2t_ref, b2_ref,
          out_ref, gu_v, gb_v, usem, bsem):
    i = pl.program_id(0)
    base = i * _GBS

    def start_one(k):
        u = uid_ref[base + k]
        b = bid_ref[base + k]
        pltpu.make_async_copy(
            utbl.at[pl.ds(u, 1)], gu_v.at[pl.ds(k, 1)], usem).start()
        pltpu.make_async_copy(
            btbl.at[pl.ds(b, 1)], gb_v.at[pl.ds(k, 1)], bsem).start()

    def wait_one():
        pltpu.make_async_copy(
            utbl.at[pl.ds(0, 1)], gu_v.at[pl.ds(0, 1)], usem).wait()
        pltpu.make_async_copy(
            btbl.at[pl.ds(0, 1)], gb_v.at[pl.ds(0, 1)], bsem).wait()

    lax.fori_loop(0, _WIN, lambda k, c: (start_one(k), c)[1], 0)

    def steady(k, c):
        wait_one()
        start_one(k + _WIN)
        return c

    lax.fori_loop(0, _GBS - _WIN, steady, 0)
    lax.fori_loop(0, _WIN, lambda k, c: (wait_one(), c)[1], 0)

    w1t = w1t_ref[:]
    h = (jnp.dot(gu_v[:], w1t[:EMBED], preferred_element_type=jnp.float32)
         + jnp.dot(gb_v[:], w1t[EMBED:], preferred_element_type=jnp.float32)
         + b1_ref[:])
    h = jnp.where(h >= 0, h, 0.01 * h)
    raw = jnp.dot(h, w2t_ref[:], preferred_element_type=jnp.float32) + b2_ref[0, 0]
    out_ref[:] = 1.0 + 4.0 * jax.nn.sigmoid(raw)


_fused = pl.pallas_call(
    _body,
    grid_spec=pltpu.PrefetchScalarGridSpec(
        num_scalar_prefetch=2,
        grid=(BATCH // _GBS,),
        in_specs=[
            pl.BlockSpec(memory_space=pl.ANY),
            pl.BlockSpec(memory_space=pl.ANY),
            pl.BlockSpec((HIDDEN, HIDDEN), lambda i, uid, bid: (0, 0)),
            pl.BlockSpec((1, HIDDEN), lambda i, uid, bid: (0, 0)),
            pl.BlockSpec((HIDDEN, 1), lambda i, uid, bid: (0, 0)),
            pl.BlockSpec(memory_space=pltpu.SMEM),
        ],
        out_specs=pl.BlockSpec((_GBS, 1), lambda i, uid, bid: (i, 0)),
        scratch_shapes=[
            pltpu.VMEM((_GBS, EMBED), jnp.float32),
            pltpu.VMEM((_GBS, EMBED), jnp.float32),
            pltpu.SemaphoreType.DMA,
            pltpu.SemaphoreType.DMA,
        ],
    ),
    out_shape=jax.ShapeDtypeStruct((BATCH, 1), jnp.float32),
)


def kernel(user_id, book_id, user_emb, book_emb, W1, b1, W2, b2):
    uid = user_id.astype(jnp.int32)
    bid = book_id.astype(jnp.int32)
    return _fused(uid, bid, user_emb, book_emb,
                  W1.swapaxes(0, 1), b1.reshape(1, HIDDEN),
                  W2.swapaxes(0, 1), b2.reshape(1, 1))
